# Initial kernel scaffold; baseline (speedup 1.0000x reference)
import jax
import jax.numpy as jnp
from jax.experimental import pallas as pl


def kernel(X, edge_index, edge_weight, W1, b1, W2, b2):
    raise NotImplementedError("write your pallas kernel here")


# speedup vs baseline: 3.6517x; 3.6517x over previous
import functools

import jax
import jax.numpy as jnp
from jax import lax
from jax.experimental import pallas as pl
from jax.experimental.pallas import tpu as pltpu
from jax.experimental.pallas import tpu_sc as plsc

N = 10000
E = 160000
D_IN = 500
D_HID = 64
D_OUT = 16

NW = 32
CH = 128
EPW = 5120
E_PAD = NW * EPW
NCHUNK = EPW // CH
N_PAD = 10240
RPT = N_PAD // 16


def _make_spmm(d):
    nvec = d // 16
    mesh = plsc.VectorSubcoreMesh(core_axis_name="c", subcore_axis_name="s")

    def body(rows_hbm, src_hbm, dst_hbm, w_hbm, out_hbm,
             src_v, dst_v, w_v, rows_v, zbuf_v, acc_sh, sem):
        c = lax.axis_index("c")
        s = lax.axis_index("s")

        def zb(i, carry):
            for k in range(nvec):
                zbuf_v[i, pl.ds(16 * k, 16)] = jnp.zeros((16,), jnp.float32)
            return carry
        lax.fori_loop(0, RPT, zb, 0)
        pltpu.sync_copy(zbuf_v, acc_sh.at[pl.ds(s * RPT, RPT)])
        plsc.subcore_barrier()

        base = (c * 16 + s) * EPW

        def chunk(j, carry):
            off = pl.multiple_of(base + j * CH, CH)
            pltpu.sync_copy(src_hbm.at[pl.ds(off, CH)], src_v)
            pltpu.sync_copy(dst_hbm.at[pl.ds(off, CH)], dst_v)
            pltpu.sync_copy(w_hbm.at[pl.ds(off, CH)], w_v)
            pltpu.async_copy(rows_hbm.at[src_v], rows_v, sem).wait()

            def mul(g, inner):
                wv = w_v[pl.ds(g * 16, 16)]
                for jj in range(16):
                    e = g * 16 + jj
                    wb = wv[jj]
                    for k in range(nvec):
                        sl = pl.ds(16 * k, 16)
                        rows_v[e, sl] = rows_v[e, sl] * wb
                return inner
            lax.fori_loop(0, CH // 16, mul, 0)

            pltpu.sync_copy(rows_v, acc_sh.at[dst_v], add=True)
            return carry
        lax.fori_loop(0, NCHUNK, chunk, 0)

        plsc.subcore_barrier()
        pltpu.sync_copy(acc_sh.at[pl.ds(s * RPT, RPT)],
                        out_hbm.at[c, pl.ds(s * RPT, RPT)])

    return pl.kernel(
        body,
        out_type=jax.ShapeDtypeStruct((2, N_PAD, d), jnp.float32),
        mesh=mesh,
        scratch_types=[
            pltpu.VMEM((CH,), jnp.int32),
            pltpu.VMEM((CH,), jnp.int32),
            pltpu.VMEM((CH,), jnp.float32),
            pltpu.VMEM((CH, d), jnp.float32),
            pltpu.VMEM((RPT, d), jnp.float32),
            pltpu.VMEM_SHARED((N_PAD, d), jnp.float32),
            pltpu.SemaphoreType.DMA,
        ],
        compiler_params=pltpu.CompilerParams(use_tc_tiling_on_sc=False),
    )


_spmm_hid = _make_spmm(D_HID)
_spmm_out = _make_spmm(D_OUT)


def _mm1(x, w1p):
    bm = 1000

    def body(x_ref, w_ref, o_ref):
        o_ref[...] = jnp.dot(x_ref[...], w_ref[...],
                             preferred_element_type=jnp.float32)

    return pl.pallas_call(
        body,
        grid=(N // bm,),
        in_specs=[pl.BlockSpec((bm, D_IN), lambda m: (m, 0)),
                  pl.BlockSpec((D_IN, D_HID), lambda m: (0, 0))],
        out_specs=pl.BlockSpec((bm, D_HID), lambda m: (m, 0)),
        out_shape=jax.ShapeDtypeStruct((N, D_HID), jnp.float32),
    )(x, w1p)


def _relu_mm2(p, b1p, w2p):
    bm = 1024

    def body(p_ref, b_ref, w_ref, o_ref):
        h = jnp.maximum(p_ref[0] + p_ref[1] + b_ref[...], 0.0)
        o_ref[...] = jnp.dot(h, w_ref[...], preferred_element_type=jnp.float32)

    return pl.pallas_call(
        body,
        grid=(N_PAD // bm,),
        in_specs=[pl.BlockSpec((2, bm, D_HID), lambda m: (0, m, 0)),
                  pl.BlockSpec((1, D_HID), lambda m: (0, 0)),
                  pl.BlockSpec((D_HID, D_OUT), lambda m: (0, 0))],
        out_specs=pl.BlockSpec((bm, D_OUT), lambda m: (m, 0)),
        out_shape=jax.ShapeDtypeStruct((N_PAD, D_OUT), jnp.float32),
    )(p, b1p, w2p)


def _final_logsoftmax(q, b2p):
    bm = 1024

    def body(q_ref, b_ref, o_ref):
        t = q_ref[0] + q_ref[1] + b_ref[...]
        col = lax.broadcasted_iota(jnp.int32, t.shape, 1)
        valid = col < 3
        m = jnp.max(jnp.where(valid, t, -1e30), axis=1, keepdims=True)
        ex = jnp.where(valid, jnp.exp(t - m), 0.0)
        lse = jnp.log(jnp.sum(ex, axis=1, keepdims=True))
        o_ref[...] = t - m - lse

    return pl.pallas_call(
        body,
        grid=(N_PAD // bm,),
        in_specs=[pl.BlockSpec((2, bm, D_OUT), lambda m: (0, m, 0)),
                  pl.BlockSpec((1, D_OUT), lambda m: (0, 0))],
        out_specs=pl.BlockSpec((bm, D_OUT), lambda m: (m, 0)),
        out_shape=jax.ShapeDtypeStruct((N_PAD, D_OUT), jnp.float32),
    )(q, b2p)


def kernel(X, edge_index, edge_weight, W1, b1, W2, b2):
    w1p = jnp.zeros((D_IN, D_HID), jnp.float32).at[:, :50].set(W1)
    b1p = jnp.zeros((1, D_HID), jnp.float32).at[0, :50].set(b1)
    w2p = jnp.zeros((D_HID, D_OUT), jnp.float32).at[:50, :3].set(W2)
    b2p = jnp.zeros((1, D_OUT), jnp.float32).at[0, :3].set(b2)

    pad = E_PAD - E
    src = jnp.concatenate([edge_index[0], jnp.zeros((pad,), jnp.int32)])
    dst = jnp.concatenate([edge_index[1], jnp.zeros((pad,), jnp.int32)])
    w = jnp.concatenate([edge_weight, jnp.zeros((pad,), jnp.float32)])

    s = _mm1(X, w1p)
    p = _spmm_hid(s, src, dst, w)
    o2 = _relu_mm2(p, b1p, w2p)
    q = _spmm_out(o2, src, dst, w)
    out16 = _final_logsoftmax(q, b2p)
    return out16[:N, :3]


# speedup vs baseline: 6.5672x; 1.7984x over previous
import functools

import jax
import jax.numpy as jnp
from jax import lax
from jax.experimental import pallas as pl
from jax.experimental.pallas import tpu as pltpu
from jax.experimental.pallas import tpu_sc as plsc

N = 10000
E = 160000
D_IN = 500
D_HID = 64
D_OUT = 16

NW = 32
CH = 128
EPW = 5120
E_PAD = NW * EPW
NCHUNK = EPW // CH
N_PAD = 10240
RPT = N_PAD // 16


NBUF = 4


def _make_spmm(d):
    nvec = d // 16
    mesh = plsc.VectorSubcoreMesh(core_axis_name="c", subcore_axis_name="s")

    def body(rows_hbm, src_hbm, dst_hbm, w_hbm, out_hbm,
             src_v, dst_v, w_v, bufs_v, zbuf_v, acc_sh, sems):
        c = lax.axis_index("c")
        s = lax.axis_index("s")
        wid = c * 16 + s

        pltpu.sync_copy(src_hbm.at[wid], src_v)
        pltpu.sync_copy(dst_hbm.at[wid], dst_v)
        pltpu.sync_copy(w_hbm.at[wid], w_v)

        def zb(i, carry):
            for k in range(nvec):
                zbuf_v[i, pl.ds(16 * k, 16)] = jnp.zeros((16,), jnp.float32)
            return carry
        lax.fori_loop(0, RPT, zb, 0)
        pltpu.sync_copy(zbuf_v, acc_sh.at[pl.ds(s * RPT, RPT)])
        plsc.subcore_barrier()

        def gather_start(j, b):
            pltpu.async_copy(rows_hbm.at[src_v.at[j]], bufs_v.at[b],
                             sems.at[b])

        def gather_wait(j, b):
            pltpu.make_async_copy(rows_hbm.at[src_v.at[j]], bufs_v.at[b],
                                  sems.at[b]).wait()

        for b in range(NBUF):
            gather_start(b, b)

        def outer(jo, carry):
            for b in range(NBUF):
                j = jo * NBUF + b
                gather_wait(j, b)

                def mul(g, inner):
                    wv = w_v[j, pl.ds(g * 16, 16)]
                    for jj in range(16):
                        e = g * 16 + jj
                        wb = wv[jj]
                        for k in range(nvec):
                            sl = pl.ds(16 * k, 16)
                            bufs_v[b, e, sl] = bufs_v[b, e, sl] * wb
                    return inner
                lax.fori_loop(0, CH // 16, mul, 0)

                pltpu.sync_copy(bufs_v.at[b], acc_sh.at[dst_v.at[j]],
                                add=True)

                @pl.when(j + NBUF < NCHUNK)
                def _():
                    gather_start(j + NBUF, b)
            return carry
        lax.fori_loop(0, NCHUNK // NBUF, outer, 0)

        plsc.subcore_barrier()
        pltpu.sync_copy(acc_sh.at[pl.ds(s * RPT, RPT)],
                        out_hbm.at[c, pl.ds(s * RPT, RPT)])

    return pl.kernel(
        body,
        out_type=jax.ShapeDtypeStruct((2, N_PAD, d), jnp.float32),
        mesh=mesh,
        scratch_types=[
            pltpu.VMEM((NCHUNK, CH), jnp.int32),
            pltpu.VMEM((NCHUNK, CH), jnp.int32),
            pltpu.VMEM((NCHUNK, CH), jnp.float32),
            pltpu.VMEM((NBUF, CH, d), jnp.float32),
            pltpu.VMEM((RPT, d), jnp.float32),
            pltpu.VMEM_SHARED((N_PAD, d), jnp.float32),
            pltpu.SemaphoreType.DMA((NBUF,)),
        ],
        compiler_params=pltpu.CompilerParams(use_tc_tiling_on_sc=False),
    )


_spmm_hid = _make_spmm(D_HID)
_spmm_out = _make_spmm(D_OUT)


def _mm1(x, w1p):
    bm = 1000

    def body(x_ref, w_ref, o_ref):
        o_ref[...] = jnp.dot(x_ref[...], w_ref[...],
                             preferred_element_type=jnp.float32)

    return pl.pallas_call(
        body,
        grid=(N // bm,),
        in_specs=[pl.BlockSpec((bm, D_IN), lambda m: (m, 0)),
                  pl.BlockSpec((D_IN, D_HID), lambda m: (0, 0))],
        out_specs=pl.BlockSpec((bm, D_HID), lambda m: (m, 0)),
        out_shape=jax.ShapeDtypeStruct((N, D_HID), jnp.float32),
    )(x, w1p)


def _relu_mm2(p, b1p, w2p):
    bm = 1024

    def body(p_ref, b_ref, w_ref, o_ref):
        h = jnp.maximum(p_ref[0] + p_ref[1] + b_ref[...], 0.0)
        o_ref[...] = jnp.dot(h, w_ref[...], preferred_element_type=jnp.float32)

    return pl.pallas_call(
        body,
        grid=(N_PAD // bm,),
        in_specs=[pl.BlockSpec((2, bm, D_HID), lambda m: (0, m, 0)),
                  pl.BlockSpec((1, D_HID), lambda m: (0, 0)),
                  pl.BlockSpec((D_HID, D_OUT), lambda m: (0, 0))],
        out_specs=pl.BlockSpec((bm, D_OUT), lambda m: (m, 0)),
        out_shape=jax.ShapeDtypeStruct((N_PAD, D_OUT), jnp.float32),
    )(p, b1p, w2p)


def _final_logsoftmax(q, b2p):
    bm = 1024

    def body(q_ref, b_ref, o_ref):
        t = q_ref[0] + q_ref[1] + b_ref[...]
        col = lax.broadcasted_iota(jnp.int32, t.shape, 1)
        valid = col < 3
        m = jnp.max(jnp.where(valid, t, -1e30), axis=1, keepdims=True)
        ex = jnp.where(valid, jnp.exp(t - m), 0.0)
        lse = jnp.log(jnp.sum(ex, axis=1, keepdims=True))
        o_ref[...] = t - m - lse

    return pl.pallas_call(
        body,
        grid=(N_PAD // bm,),
        in_specs=[pl.BlockSpec((2, bm, D_OUT), lambda m: (0, m, 0)),
                  pl.BlockSpec((1, D_OUT), lambda m: (0, 0))],
        out_specs=pl.BlockSpec((bm, D_OUT), lambda m: (m, 0)),
        out_shape=jax.ShapeDtypeStruct((N_PAD, D_OUT), jnp.float32),
    )(q, b2p)


def kernel(X, edge_index, edge_weight, W1, b1, W2, b2):
    w1p = jnp.zeros((D_IN, D_HID), jnp.float32).at[:, :50].set(W1)
    b1p = jnp.zeros((1, D_HID), jnp.float32).at[0, :50].set(b1)
    w2p = jnp.zeros((D_HID, D_OUT), jnp.float32).at[:50, :3].set(W2)
    b2p = jnp.zeros((1, D_OUT), jnp.float32).at[0, :3].set(b2)

    pad = E_PAD - E
    src = jnp.concatenate([edge_index[0], jnp.zeros((pad,), jnp.int32)])
    dst = jnp.concatenate([edge_index[1], jnp.zeros((pad,), jnp.int32)])
    w = jnp.concatenate([edge_weight, jnp.zeros((pad,), jnp.float32)])
    src = src.reshape(NW, NCHUNK, CH)
    dst = dst.reshape(NW, NCHUNK, CH)
    w = w.reshape(NW, NCHUNK, CH)

    s = _mm1(X, w1p)
    p = _spmm_hid(s, src, dst, w)
    o2 = _relu_mm2(p, b1p, w2p)
    q = _spmm_out(o2, src, dst, w)
    out16 = _final_logsoftmax(q, b2p)
    return out16[:N, :3]


# speedup vs baseline: 6.7245x; 1.0239x over previous
import functools

import jax
import jax.numpy as jnp
from jax import lax
from jax.experimental import pallas as pl
from jax.experimental.pallas import tpu as pltpu
from jax.experimental.pallas import tpu_sc as plsc

N = 10000
E = 160000
D_IN = 500
D_HID = 64
D_OUT = 16

NW = 32
CH = 128
EPW = 5120
E_PAD = NW * EPW
NCHUNK = EPW // CH
N_PAD = 10240
RPT = N_PAD // 16


NBUF = 4


def _make_spmm(d):
    nvec = d // 16
    mesh = plsc.VectorSubcoreMesh(core_axis_name="c", subcore_axis_name="s")

    def body(rows_hbm, src_hbm, dst_hbm, w_hbm, out_hbm,
             src_v, dst_v, w_v, bufs_v, zbuf_v, acc_sh, gsems, ssems, isems):
        c = lax.axis_index("c")
        s = lax.axis_index("s")
        wid = c * 16 + s

        h_src = pltpu.async_copy(src_hbm.at[wid], src_v, isems.at[0])
        h_dst = pltpu.async_copy(dst_hbm.at[wid], dst_v, isems.at[1])
        h_w = pltpu.async_copy(w_hbm.at[wid], w_v, isems.at[2])

        def zb(i, carry):
            for k in range(nvec):
                zbuf_v[i, pl.ds(16 * k, 16)] = jnp.zeros((16,), jnp.float32)
            return carry
        lax.fori_loop(0, RPT, zb, 0)
        pltpu.sync_copy(zbuf_v, acc_sh.at[pl.ds(s * RPT, RPT)])
        h_src.wait()
        h_dst.wait()
        h_w.wait()
        plsc.subcore_barrier()

        def gather_start(j, b):
            pltpu.async_copy(rows_hbm.at[src_v.at[j]], bufs_v.at[b],
                             gsems.at[b])

        def gather_wait(j, b):
            pltpu.make_async_copy(rows_hbm.at[src_v.at[j]], bufs_v.at[b],
                                  gsems.at[b]).wait()

        def scatter_wait(j, b):
            pltpu.make_async_copy(bufs_v.at[b], acc_sh.at[dst_v.at[j]],
                                  ssems.at[b]).wait()

        for b in range(NBUF - 1):
            gather_start(b, b)

        def outer(jo, carry):
            for b in range(NBUF):
                j = jo * NBUF + b
                gather_wait(j, b)

                def mul(g, inner):
                    wv = w_v[j, pl.ds(g * 16, 16)]
                    for jj in range(16):
                        e = g * 16 + jj
                        wb = wv[jj]
                        for k in range(nvec):
                            sl = pl.ds(16 * k, 16)
                            bufs_v[b, e, sl] = bufs_v[b, e, sl] * wb
                    return inner
                lax.fori_loop(0, CH // 16, mul, 0)

                pltpu.async_copy(bufs_v.at[b], acc_sh.at[dst_v.at[j]],
                                 ssems.at[b], add=True)

                bprev = (b - 1) % NBUF
                if b == 0:
                    @pl.when(jo > 0)
                    def _():
                        scatter_wait(j - 1, bprev)
                else:
                    scatter_wait(j - 1, bprev)

                @pl.when(j + NBUF - 1 < NCHUNK)
                def _():
                    gather_start(j + NBUF - 1, bprev)
            return carry
        lax.fori_loop(0, NCHUNK // NBUF, outer, 0)

        scatter_wait(NCHUNK - 1, (NCHUNK - 1) % NBUF)
        plsc.subcore_barrier()
        pltpu.sync_copy(acc_sh.at[pl.ds(s * RPT, RPT)],
                        out_hbm.at[c, pl.ds(s * RPT, RPT)])

    return pl.kernel(
        body,
        out_type=jax.ShapeDtypeStruct((2, N_PAD, d), jnp.float32),
        mesh=mesh,
        scratch_types=[
            pltpu.VMEM((NCHUNK, CH), jnp.int32),
            pltpu.VMEM((NCHUNK, CH), jnp.int32),
            pltpu.VMEM((NCHUNK, CH), jnp.float32),
            pltpu.VMEM((NBUF, CH, d), jnp.float32),
            pltpu.VMEM((RPT, d), jnp.float32),
            pltpu.VMEM_SHARED((N_PAD, d), jnp.float32),
            pltpu.SemaphoreType.DMA((NBUF,)),
            pltpu.SemaphoreType.DMA((NBUF,)),
            pltpu.SemaphoreType.DMA((3,)),
        ],
        compiler_params=pltpu.CompilerParams(use_tc_tiling_on_sc=False),
    )


_spmm_hid = _make_spmm(D_HID)
_spmm_out = _make_spmm(D_OUT)


def _mm1(x, w1p):
    bm = 1000

    def body(x_ref, w_ref, o_ref):
        o_ref[...] = jnp.dot(x_ref[...], w_ref[...],
                             preferred_element_type=jnp.float32)

    return pl.pallas_call(
        body,
        grid=(N // bm,),
        in_specs=[pl.BlockSpec((bm, D_IN), lambda m: (m, 0)),
                  pl.BlockSpec((D_IN, D_HID), lambda m: (0, 0))],
        out_specs=pl.BlockSpec((bm, D_HID), lambda m: (m, 0)),
        out_shape=jax.ShapeDtypeStruct((N, D_HID), jnp.float32),
    )(x, w1p)


def _relu_mm2(p, b1p, w2p):
    bm = 1024

    def body(p_ref, b_ref, w_ref, o_ref):
        h = jnp.maximum(p_ref[0] + p_ref[1] + b_ref[...], 0.0)
        o_ref[...] = jnp.dot(h, w_ref[...], preferred_element_type=jnp.float32)

    return pl.pallas_call(
        body,
        grid=(N_PAD // bm,),
        in_specs=[pl.BlockSpec((2, bm, D_HID), lambda m: (0, m, 0)),
                  pl.BlockSpec((1, D_HID), lambda m: (0, 0)),
                  pl.BlockSpec((D_HID, D_OUT), lambda m: (0, 0))],
        out_specs=pl.BlockSpec((bm, D_OUT), lambda m: (m, 0)),
        out_shape=jax.ShapeDtypeStruct((N_PAD, D_OUT), jnp.float32),
    )(p, b1p, w2p)


def _final_logsoftmax(q, b2p):
    bm = 1000

    def body(q_ref, b_ref, o_ref):
        t = q_ref[0] + q_ref[1] + b_ref[...]
        col = lax.broadcasted_iota(jnp.int32, t.shape, 1)
        valid = col < 3
        m = jnp.max(jnp.where(valid, t, -1e30), axis=1, keepdims=True)
        ex = jnp.where(valid, jnp.exp(t - m), 0.0)
        lse = jnp.log(jnp.sum(ex, axis=1, keepdims=True))
        o_ref[...] = (t - m - lse)[:, :3]

    return pl.pallas_call(
        body,
        grid=(N // bm,),
        in_specs=[pl.BlockSpec((2, bm, D_OUT), lambda m: (0, m, 0)),
                  pl.BlockSpec((1, D_OUT), lambda m: (0, 0))],
        out_specs=pl.BlockSpec((bm, 3), lambda m: (m, 0)),
        out_shape=jax.ShapeDtypeStruct((N, 3), jnp.float32),
    )(q, b2p)


def kernel(X, edge_index, edge_weight, W1, b1, W2, b2):
    w1p = jnp.zeros((D_IN, D_HID), jnp.float32).at[:, :50].set(W1)
    b1p = jnp.zeros((1, D_HID), jnp.float32).at[0, :50].set(b1)
    w2p = jnp.zeros((D_HID, D_OUT), jnp.float32).at[:50, :3].set(W2)
    b2p = jnp.zeros((1, D_OUT), jnp.float32).at[0, :3].set(b2)

    pad = E_PAD - E
    src = jnp.concatenate([edge_index[0], jnp.zeros((pad,), jnp.int32)])
    dst = jnp.concatenate([edge_index[1], jnp.zeros((pad,), jnp.int32)])
    w = jnp.concatenate([edge_weight, jnp.zeros((pad,), jnp.float32)])
    src = src.reshape(NW, NCHUNK, CH)
    dst = dst.reshape(NW, NCHUNK, CH)
    w = w.reshape(NW, NCHUNK, CH)

    s = _mm1(X, w1p)
    p = _spmm_hid(s, src, dst, w)
    o2 = _relu_mm2(p, b1p, w2p)
    q = _spmm_out(o2, src, dst, w)
    return _final_logsoftmax(q, b2p)


# speedup vs baseline: 9.1047x; 1.3540x over previous
import jax
import jax.numpy as jnp
from jax import lax
from jax.experimental import pallas as pl
from jax.experimental.pallas import tpu as pltpu
from jax.experimental.pallas import tpu_sc as plsc

N = 10000
E = 160000
D_IN = 500
D_HID = 64
D_OUT = 16

NW = 32
CH = 128
EPW = 5120
E_PAD = NW * EPW
NCHUNK = EPW // CH
N_PAD = 10112
RPT = N_PAD // 16

NBUF = 4


def _make_spmm(d, nsrc):
    dh = min(d, 32)
    nh = d // dh
    nvec = dh // 16
    mesh = plsc.VectorSubcoreMesh(core_axis_name="c", subcore_axis_name="s")

    def impl(tabs_hbm, src_hbm, dst_hbm, w_hbm, out_hbm,
             src_v, dst_v, w_v, bufs_v, zbuf_v, tab_sh, acc_sh,
             gsems, ssems, isems):
        c = lax.axis_index("c")
        s = lax.axis_index("s")
        wid = c * 16 + s

        h_src = pltpu.async_copy(src_hbm.at[wid], src_v, isems.at[0])
        h_dst = pltpu.async_copy(dst_hbm.at[wid], dst_v, isems.at[1])
        h_w = pltpu.async_copy(w_hbm.at[wid], w_v, isems.at[2])
        h_src.wait()
        h_dst.wait()
        h_w.wait()

        def gather_start(j, b):
            pltpu.async_copy(tab_sh.at[src_v.at[j]], bufs_v.at[b],
                             gsems.at[b])

        def gather_wait(j, b):
            pltpu.make_async_copy(tab_sh.at[src_v.at[j]], bufs_v.at[b],
                                  gsems.at[b]).wait()

        def scatter_wait(j, b):
            pltpu.make_async_copy(bufs_v.at[b], acc_sh.at[dst_v.at[j]],
                                  ssems.at[b]).wait()

        for h in range(nh):
            if nsrc % 128 == 0:
                rpt_s = nsrc // 16
                h_tab = pltpu.async_copy(
                    tabs_hbm[h].at[pl.ds(s * rpt_s, rpt_s)],
                    tab_sh.at[pl.ds(s * rpt_s, rpt_s)], isems.at[3])

            def zb(i, carry):
                for k in range(nvec):
                    zbuf_v[i, pl.ds(16 * k, 16)] = jnp.zeros((16,),
                                                             jnp.float32)
                return carry
            lax.fori_loop(0, RPT, zb, 0)

            if nsrc % 128 == 0:
                h_tab.wait()
            else:
                full = (nsrc // 16 + 8) // 8 * 8
                rem = nsrc - 15 * full

                @pl.when(s < 15)
                def _():
                    pltpu.async_copy(tabs_hbm[h].at[pl.ds(s * full, full)],
                                     tab_sh.at[pl.ds(s * full, full)],
                                     isems.at[3]).wait()

                @pl.when(s == 15)
                def _():
                    pltpu.async_copy(tabs_hbm[h].at[pl.ds(15 * full, rem)],
                                     tab_sh.at[pl.ds(15 * full, rem)],
                                     isems.at[3]).wait()
            pltpu.sync_copy(zbuf_v, acc_sh.at[pl.ds(s * RPT, RPT)])
            plsc.subcore_barrier()

            for b in range(NBUF - 1):
                gather_start(b, b)

            def outer(jo, carry):
                for b in range(NBUF):
                    j = jo * NBUF + b
                    gather_wait(j, b)

                    def mul(g, inner):
                        wv = w_v[j, pl.ds(g * 16, 16)]
                        for jj in range(16):
                            e = g * 16 + jj
                            wb = wv[jj]
                            for k in range(nvec):
                                sl = pl.ds(16 * k, 16)
                                bufs_v[b, e, sl] = bufs_v[b, e, sl] * wb
                        return inner
                    lax.fori_loop(0, CH // 16, mul, 0)

                    pltpu.async_copy(bufs_v.at[b], acc_sh.at[dst_v.at[j]],
                                     ssems.at[b], add=True)

                    bprev = (b - 1) % NBUF
                    if b == 0:
                        @pl.when(jo > 0)
                        def _():
                            scatter_wait(j - 1, bprev)
                    else:
                        scatter_wait(j - 1, bprev)

                    @pl.when(j + NBUF - 1 < NCHUNK)
                    def _():
                        gather_start(j + NBUF - 1, bprev)
                return carry
            lax.fori_loop(0, NCHUNK // NBUF, outer, 0)

            scatter_wait(NCHUNK - 1, (NCHUNK - 1) % NBUF)
            plsc.subcore_barrier()
            pltpu.sync_copy(acc_sh.at[pl.ds(s * RPT, RPT)],
                            out_hbm.at[c, h, pl.ds(s * RPT, RPT)])

    if nh == 2:
        def body(tab_a, tab_b, src_hbm, dst_hbm, w_hbm, out_hbm, *scr):
            impl([tab_a, tab_b], src_hbm, dst_hbm, w_hbm, out_hbm, *scr)
    else:
        def body(tab_a, src_hbm, dst_hbm, w_hbm, out_hbm, *scr):
            impl([tab_a], src_hbm, dst_hbm, w_hbm, out_hbm, *scr)

    return pl.kernel(
        body,
        out_type=jax.ShapeDtypeStruct((2, nh, N_PAD, dh), jnp.float32),
        mesh=mesh,
        scratch_types=[
            pltpu.VMEM((NCHUNK, CH), jnp.int32),
            pltpu.VMEM((NCHUNK, CH), jnp.int32),
            pltpu.VMEM((NCHUNK, CH), jnp.float32),
            pltpu.VMEM((NBUF, CH, dh), jnp.float32),
            pltpu.VMEM((RPT, dh), jnp.float32),
            pltpu.VMEM_SHARED((nsrc, dh), jnp.float32),
            pltpu.VMEM_SHARED((N_PAD, dh), jnp.float32),
            pltpu.SemaphoreType.DMA((NBUF,)),
            pltpu.SemaphoreType.DMA((NBUF,)),
            pltpu.SemaphoreType.DMA((4,)),
        ],
        compiler_params=pltpu.CompilerParams(use_tc_tiling_on_sc=False),
    )


_spmm_hid = _make_spmm(D_HID, N)
_spmm_out = _make_spmm(D_OUT, N_PAD)


def _mm1(x, w1p):
    bm = 1000

    def body(x_ref, w_ref, oa_ref, ob_ref):
        r = jnp.dot(x_ref[...], w_ref[...], preferred_element_type=jnp.float32)
        oa_ref[...] = r[:, :32]
        ob_ref[...] = r[:, 32:]

    return pl.pallas_call(
        body,
        grid=(N // bm,),
        in_specs=[pl.BlockSpec((bm, D_IN), lambda m: (m, 0)),
                  pl.BlockSpec((D_IN, D_HID), lambda m: (0, 0))],
        out_specs=[pl.BlockSpec((bm, 32), lambda m: (m, 0)),
                   pl.BlockSpec((bm, 32), lambda m: (m, 0))],
        out_shape=[jax.ShapeDtypeStruct((N, 32), jnp.float32),
                   jax.ShapeDtypeStruct((N, 32), jnp.float32)],
    )(x, w1p)


def _relu_mm2(p, b1p, w2p):
    bm = 632

    def body(p_ref, b_ref, w_ref, o_ref):
        hcat = jnp.concatenate(
            [p_ref[0, 0] + p_ref[1, 0], p_ref[0, 1] + p_ref[1, 1]], axis=1)
        h = jnp.maximum(hcat + b_ref[...], 0.0)
        o_ref[...] = jnp.dot(h, w_ref[...], preferred_element_type=jnp.float32)

    return pl.pallas_call(
        body,
        grid=(N_PAD // bm,),
        in_specs=[pl.BlockSpec((2, 2, bm, 32), lambda m: (0, 0, m, 0)),
                  pl.BlockSpec((1, D_HID), lambda m: (0, 0)),
                  pl.BlockSpec((D_HID, D_OUT), lambda m: (0, 0))],
        out_specs=pl.BlockSpec((bm, D_OUT), lambda m: (m, 0)),
        out_shape=jax.ShapeDtypeStruct((N_PAD, D_OUT), jnp.float32),
    )(p, b1p, w2p)


def _final_logsoftmax(q, b2p):
    bm = 1000

    def body(q_ref, b_ref, o_ref):
        t = q_ref[0, 0] + q_ref[1, 0] + b_ref[...]
        col = lax.broadcasted_iota(jnp.int32, t.shape, 1)
        valid = col < 3
        m = jnp.max(jnp.where(valid, t, -1e30), axis=1, keepdims=True)
        ex = jnp.where(valid, jnp.exp(t - m), 0.0)
        lse = jnp.log(jnp.sum(ex, axis=1, keepdims=True))
        o_ref[...] = (t - m - lse)[:, :3]

    return pl.pallas_call(
        body,
        grid=(N // bm,),
        in_specs=[pl.BlockSpec((2, 1, bm, D_OUT), lambda m: (0, 0, m, 0)),
                  pl.BlockSpec((1, D_OUT), lambda m: (0, 0))],
        out_specs=pl.BlockSpec((bm, 3), lambda m: (m, 0)),
        out_shape=jax.ShapeDtypeStruct((N, 3), jnp.float32),
    )(q, b2p)


def kernel(X, edge_index, edge_weight, W1, b1, W2, b2):
    w1p = jnp.zeros((D_IN, D_HID), jnp.float32).at[:, :50].set(W1)
    b1p = jnp.zeros((1, D_HID), jnp.float32).at[0, :50].set(b1)
    w2p = jnp.zeros((D_HID, D_OUT), jnp.float32).at[:50, :3].set(W2)
    b2p = jnp.zeros((1, D_OUT), jnp.float32).at[0, :3].set(b2)

    pad = E_PAD - E
    src = jnp.concatenate([edge_index[0], jnp.zeros((pad,), jnp.int32)])
    dst = jnp.concatenate([edge_index[1], jnp.zeros((pad,), jnp.int32)])
    w = jnp.concatenate([edge_weight, jnp.zeros((pad,), jnp.float32)])
    src = src.reshape(NW, NCHUNK, CH)
    dst = dst.reshape(NW, NCHUNK, CH)
    w = w.reshape(NW, NCHUNK, CH)

    s_a, s_b = _mm1(X, w1p)
    p = _spmm_hid(s_a, s_b, src, dst, w)
    o2 = _relu_mm2(p, b1p, w2p)
    q = _spmm_out(o2, src, dst, w)
    return _final_logsoftmax(q, b2p)


# speedup vs baseline: 10.0363x; 1.1023x over previous
import jax
import jax.numpy as jnp
from jax import lax
from jax.experimental import pallas as pl
from jax.experimental.pallas import tpu as pltpu
from jax.experimental.pallas import tpu_sc as plsc

N = 10000
E = 160000
D_IN = 500
D_HID = 64
D_OUT = 16

NW = 32
CH = 128
EPW = 5120
E_PAD = NW * EPW
NCHUNK = EPW // CH
N_PAD = 10112
RPT = N_PAD // 16

NBUF = 4


def _make_spmm(d, nsrc):
    dh = min(d, 32)
    nh = d // dh
    nvec = dh // 16
    mesh = plsc.VectorSubcoreMesh(core_axis_name="c", subcore_axis_name="s")

    def impl(tabs_hbm, src_hbm, dst_hbm, w_hbm, out_hbm,
             src_v, dst_v, w_v, bufs_v, zbuf_v, tab_sh, acc_sh,
             gsems, ssems, isems):
        c = lax.axis_index("c")
        s = lax.axis_index("s")
        wid = c * 16 + s

        h_src = pltpu.async_copy(src_hbm.at[wid], src_v, isems.at[0])
        h_dst = pltpu.async_copy(dst_hbm.at[wid], dst_v, isems.at[1])
        h_w = pltpu.async_copy(w_hbm.at[wid], w_v, isems.at[2])
        h_src.wait()
        h_dst.wait()
        h_w.wait()

        def gather_start(j, b):
            pltpu.async_copy(tab_sh.at[src_v.at[j]], bufs_v.at[b],
                             gsems.at[b])

        def gather_wait(j, b):
            pltpu.make_async_copy(tab_sh.at[src_v.at[j]], bufs_v.at[b],
                                  gsems.at[b]).wait()

        def scatter_wait(j, b):
            pltpu.make_async_copy(bufs_v.at[b], acc_sh.at[dst_v.at[j]],
                                  ssems.at[b]).wait()

        for h in range(nh):
            if nsrc % 128 == 0:
                rpt_s = nsrc // 16
                h_tab = pltpu.async_copy(
                    tabs_hbm[h].at[pl.ds(s * rpt_s, rpt_s)],
                    tab_sh.at[pl.ds(s * rpt_s, rpt_s)], isems.at[3])

            def zb(i, carry):
                for k in range(nvec):
                    zbuf_v[i, pl.ds(16 * k, 16)] = jnp.zeros((16,),
                                                             jnp.float32)
                return carry
            lax.fori_loop(0, RPT, zb, 0)

            if nsrc % 128 == 0:
                h_tab.wait()
            else:
                full = (nsrc // 16 + 8) // 8 * 8
                rem = nsrc - 15 * full

                @pl.when(s < 15)
                def _():
                    pltpu.async_copy(tabs_hbm[h].at[pl.ds(s * full, full)],
                                     tab_sh.at[pl.ds(s * full, full)],
                                     isems.at[3]).wait()

                @pl.when(s == 15)
                def _():
                    pltpu.async_copy(tabs_hbm[h].at[pl.ds(15 * full, rem)],
                                     tab_sh.at[pl.ds(15 * full, rem)],
                                     isems.at[3]).wait()
            pltpu.sync_copy(zbuf_v, acc_sh.at[pl.ds(s * RPT, RPT)])
            plsc.subcore_barrier()

            for b in range(NBUF - 1):
                gather_start(b, b)

            def outer(jo, carry):
                for b in range(NBUF):
                    j = jo * NBUF + b
                    gather_wait(j, b)

                    def mul(g, inner):
                        wv = w_v[j, pl.ds(g * 16, 16)]
                        for jj in range(16):
                            e = g * 16 + jj
                            wb = wv[jj]
                            for k in range(nvec):
                                sl = pl.ds(16 * k, 16)
                                bufs_v[b, e, sl] = bufs_v[b, e, sl] * wb
                        return inner
                    lax.fori_loop(0, CH // 16, mul, 0)

                    pltpu.async_copy(bufs_v.at[b], acc_sh.at[dst_v.at[j]],
                                     ssems.at[b], add=True)

                    bprev = (b - 1) % NBUF
                    if b == 0:
                        @pl.when(jo > 0)
                        def _():
                            scatter_wait(j - 1, bprev)
                    else:
                        scatter_wait(j - 1, bprev)

                    @pl.when(j + NBUF - 1 < NCHUNK)
                    def _():
                        gather_start(j + NBUF - 1, bprev)
                return carry
            lax.fori_loop(0, NCHUNK // NBUF, outer, 0)

            scatter_wait(NCHUNK - 1, (NCHUNK - 1) % NBUF)
            plsc.subcore_barrier()
            pltpu.sync_copy(acc_sh.at[pl.ds(s * RPT, RPT)],
                            out_hbm.at[c, h, pl.ds(s * RPT, RPT)])

    if nh == 2:
        def body(tab2_hbm, src_hbm, dst_hbm, w_hbm, out_hbm, *scr):
            impl([tab2_hbm.at[0], tab2_hbm.at[1]],
                 src_hbm, dst_hbm, w_hbm, out_hbm, *scr)
    else:
        def body(tab_a, src_hbm, dst_hbm, w_hbm, out_hbm, *scr):
            impl([tab_a], src_hbm, dst_hbm, w_hbm, out_hbm, *scr)

    return pl.kernel(
        body,
        out_type=jax.ShapeDtypeStruct((2, nh, N_PAD, dh), jnp.float32),
        mesh=mesh,
        scratch_types=[
            pltpu.VMEM((NCHUNK, CH), jnp.int32),
            pltpu.VMEM((NCHUNK, CH), jnp.int32),
            pltpu.VMEM((NCHUNK, CH), jnp.float32),
            pltpu.VMEM((NBUF, CH, dh), jnp.float32),
            pltpu.VMEM((RPT, dh), jnp.float32),
            pltpu.VMEM_SHARED((nsrc, dh), jnp.float32),
            pltpu.VMEM_SHARED((N_PAD, dh), jnp.float32),
            pltpu.SemaphoreType.DMA((NBUF,)),
            pltpu.SemaphoreType.DMA((NBUF,)),
            pltpu.SemaphoreType.DMA((4,)),
        ],
        compiler_params=pltpu.CompilerParams(use_tc_tiling_on_sc=False),
    )


_spmm_hid = _make_spmm(D_HID, N)
_spmm_out = _make_spmm(D_OUT, N_PAD)


def _mm1(xt, w1p):
    bm = 1024

    def body(x_ref, w_ref, o_ref):
        r = lax.dot_general(x_ref[...], w_ref[...],
                            dimension_numbers=(((0,), (0,)), ((), ())),
                            preferred_element_type=jnp.float32)
        o_ref[0] = r[:, :32]
        o_ref[1] = r[:, 32:]

    return pl.pallas_call(
        body,
        grid=((N + bm - 1) // bm,),
        in_specs=[pl.BlockSpec((D_IN, bm), lambda m: (0, m)),
                  pl.BlockSpec((D_IN, D_HID), lambda m: (0, 0))],
        out_specs=pl.BlockSpec((2, bm, 32), lambda m: (0, m, 0)),
        out_shape=jax.ShapeDtypeStruct((2, N, 32), jnp.float32),
    )(xt, w1p)


def _relu_mm2(p, b1p, w2p):
    bm = 632

    def body(p_ref, b_ref, w_ref, o_ref):
        hcat = jnp.concatenate(
            [p_ref[0, 0] + p_ref[1, 0], p_ref[0, 1] + p_ref[1, 1]], axis=1)
        h = jnp.maximum(hcat + b_ref[...], 0.0)
        o_ref[...] = jnp.dot(h, w_ref[...], preferred_element_type=jnp.float32)

    return pl.pallas_call(
        body,
        grid=(N_PAD // bm,),
        in_specs=[pl.BlockSpec((2, 2, bm, 32), lambda m: (0, 0, m, 0)),
                  pl.BlockSpec((1, D_HID), lambda m: (0, 0)),
                  pl.BlockSpec((D_HID, D_OUT), lambda m: (0, 0))],
        out_specs=pl.BlockSpec((bm, D_OUT), lambda m: (m, 0)),
        out_shape=jax.ShapeDtypeStruct((N_PAD, D_OUT), jnp.float32),
    )(p, b1p, w2p)


def _final_logsoftmax(q, b2p):
    bm = 1000

    def body(q_ref, b_ref, o_ref):
        t = q_ref[0, 0] + q_ref[1, 0] + b_ref[...]
        col = lax.broadcasted_iota(jnp.int32, t.shape, 1)
        valid = col < 3
        m = jnp.max(jnp.where(valid, t, -1e30), axis=1, keepdims=True)
        ex = jnp.where(valid, jnp.exp(t - m), 0.0)
        lse = jnp.log(jnp.sum(ex, axis=1, keepdims=True))
        o_ref[...] = (t - m - lse)[:, :3]

    return pl.pallas_call(
        body,
        grid=(N // bm,),
        in_specs=[pl.BlockSpec((2, 1, bm, D_OUT), lambda m: (0, 0, m, 0)),
                  pl.BlockSpec((1, D_OUT), lambda m: (0, 0))],
        out_specs=pl.BlockSpec((bm, 3), lambda m: (m, 0)),
        out_shape=jax.ShapeDtypeStruct((N, 3), jnp.float32),
    )(q, b2p)


def kernel(X, edge_index, edge_weight, W1, b1, W2, b2):
    w1p = jnp.zeros((D_IN, D_HID), jnp.float32).at[:, :50].set(W1)
    b1p = jnp.zeros((1, D_HID), jnp.float32).at[0, :50].set(b1)
    w2p = jnp.zeros((D_HID, D_OUT), jnp.float32).at[:50, :3].set(W2)
    b2p = jnp.zeros((1, D_OUT), jnp.float32).at[0, :3].set(b2)

    pad = E_PAD - E
    src = jnp.concatenate([edge_index[0], jnp.zeros((pad,), jnp.int32)])
    dst = jnp.concatenate([edge_index[1], jnp.zeros((pad,), jnp.int32)])
    w = jnp.concatenate([edge_weight, jnp.zeros((pad,), jnp.float32)])
    src = src.reshape(NW, NCHUNK, CH)
    dst = dst.reshape(NW, NCHUNK, CH)
    w = w.reshape(NW, NCHUNK, CH)

    s2 = _mm1(X.T, w1p)
    p = _spmm_hid(s2, src, dst, w)
    o2 = _relu_mm2(p, b1p, w2p)
    q = _spmm_out(o2, src, dst, w)
    return _final_logsoftmax(q, b2p)
